# baseline (speedup 1.0000x reference)
import functools

import jax
import jax.numpy as jnp
from jax import lax
from jax.experimental import pallas as pl
from jax.experimental.pallas import tpu as pltpu
from jax.experimental.pallas import tpu_sc as plsc

_N_CAT = 26
_N_CONT = 13
_VOCAB_P1 = 100001
_CAT_DIM = 16
_CONT_DIM = 16

_NC = 2
_NS = 16
_NW = _NC * _NS

_VMAIN = 99968
_NTAIL = _VOCAB_P1 - _VMAIN
_TAILW = 48


def _make_detiler():
    n_tasks = _N_CAT * 50

    mesh = plsc.VectorSubcoreMesh(core_axis_name="c", subcore_axis_name="s")

    @functools.partial(
        pl.kernel,
        mesh=mesh,
        out_type=jax.ShapeDtypeStruct((_N_CAT * _VOCAB_P1 * _CAT_DIM,),
                                      jnp.float32),
        compiler_params=pltpu.CompilerParams(
            use_tc_tiling_on_sc=True, needs_layout_passes=False),
        scratch_types=[
            pltpu.VMEM((_CAT_DIM, 2048), jnp.float32),
            pltpu.VMEM((2048 * _CAT_DIM,), jnp.float32),
            pltpu.VMEM((_CAT_DIM, _TAILW), jnp.float32),
        ],
    )
    def detile(t3_hbm, tail_hbm, out_hbm, buf, tbuf, lbuf):
        wid = lax.axis_index("s") * _NC + lax.axis_index("c")
        lane = lax.iota(jnp.int32, 16)

        def transpose_strip(src, n_v):
            def tr_body(g, c):
                v0 = g * 16
                for k in range(16):
                    vec = plsc.load_gather(
                        src, [lane, jnp.full((16,), v0 + k, jnp.int32)])
                    tbuf[pl.ds((v0 + k) * 16, 16)] = vec
                return c

            lax.fori_loop(0, n_v // 16, tr_body, 0)

        def task_body(k, carry):
            t = wid + k * _NW
            i = t // 50
            j = t % 50

            @pl.when((t < n_tasks) & (j < 48))
            def _strip():
                v0 = pl.multiple_of(j * 2048, 128)
                pltpu.sync_copy(t3_hbm.at[i, :, pl.ds(v0, 2048)], buf)
                transpose_strip(buf, 2048)
                pltpu.sync_copy(
                    tbuf,
                    out_hbm.at[pl.ds(
                        pl.multiple_of((i * _VOCAB_P1 + v0) * _CAT_DIM, 16),
                        2048 * _CAT_DIM)])

            @pl.when((t < n_tasks) & (j == 48))
            def _strip_last():
                v0 = pl.multiple_of(98304, 128)
                pltpu.sync_copy(
                    t3_hbm.at[i, :, pl.ds(v0, 1664)],
                    buf.at[:, pl.ds(0, 1664)])
                transpose_strip(buf, 1664)
                pltpu.sync_copy(
                    tbuf.at[pl.ds(0, 1664 * _CAT_DIM)],
                    out_hbm.at[pl.ds(
                        pl.multiple_of((i * _VOCAB_P1 + v0) * _CAT_DIM, 16),
                        1664 * _CAT_DIM)])

            @pl.when((t < n_tasks) & (j == 49))
            def _tail():
                pltpu.sync_copy(tail_hbm.at[i], lbuf)
                transpose_strip(lbuf, _TAILW)
                pltpu.sync_copy(
                    tbuf.at[pl.ds(0, _NTAIL * _CAT_DIM)],
                    out_hbm.at[pl.ds(
                        pl.multiple_of(
                            (i * _VOCAB_P1 + _VMAIN) * _CAT_DIM, 16),
                        _NTAIL * _CAT_DIM)])

            return carry

        lax.fori_loop(0, pl.cdiv(n_tasks, _NW), task_body, 0)

    return detile


def _make_cat_gather(b: int):
    chunk = b // _NW

    mesh = plsc.VectorSubcoreMesh(core_axis_name="c", subcore_axis_name="s")

    @functools.partial(
        pl.kernel,
        mesh=mesh,
        out_type=jax.ShapeDtypeStruct((_N_CAT * _CAT_DIM, b), jnp.float32),
        compiler_params=pltpu.CompilerParams(
            use_tc_tiling_on_sc=False, needs_layout_passes=False),
        scratch_types=[
            pltpu.VMEM((chunk,), jnp.float32),
            pltpu.VMEM((chunk,), jnp.int32),
            pltpu.VMEM((chunk, _CAT_DIM), jnp.float32),
            pltpu.VMEM((_CAT_DIM, chunk), jnp.float32),
            pltpu.SemaphoreType.DMA,
        ],
    )
    def cat_gather(xt_hbm, t16_hbm, out_hbm, idxf_v, idx_v, val_v, wbuf, sem):
        wid = lax.axis_index("s") * _NC + lax.axis_index("c")
        b0 = wid * chunk
        lane = lax.iota(jnp.int32, 16)

        def col_body(i, carry):
            pltpu.sync_copy(xt_hbm.at[i, pl.ds(b0, chunk)], idxf_v)
            base = i * _VOCAB_P1

            def cvt_body(u, c):
                sl = pl.ds(u * 16, 16)
                idx_v[sl] = idxf_v[sl].astype(jnp.int32) + base
                return c

            lax.fori_loop(0, chunk // 16, cvt_body, 0)
            pltpu.async_copy(t16_hbm.at[idx_v], val_v, sem).wait()

            def tr_body(g, c):
                n0 = g * 16
                for d in range(_CAT_DIM):
                    vec = plsc.load_gather(
                        val_v, [n0 + lane, jnp.full((16,), d, jnp.int32)])
                    wbuf[d, pl.ds(n0, 16)] = vec
                return c

            lax.fori_loop(0, chunk // 16, tr_body, 0)
            pltpu.sync_copy(
                wbuf,
                out_hbm.at[pl.ds(i * _CAT_DIM, _CAT_DIM), pl.ds(b0, chunk)])
            return carry

        lax.fori_loop(0, _N_CAT, col_body, 0)

    return cat_gather


def _cont_body(x_ref, w_ref, o_ref):
    for s in range(_N_CONT):
        o_ref[s * _CONT_DIM:(s + 1) * _CONT_DIM, :] = (
            w_ref[s, :][:, None] * x_ref[_N_CAT + s, :][None, :]
        )


def _cont_embed_t(xt, cont_w):
    b = xt.shape[1]
    nb = 1024
    grid = (b // nb,)
    return pl.pallas_call(
        _cont_body,
        grid=grid,
        in_specs=[
            pl.BlockSpec((_N_CAT + _N_CONT, nb), lambda j: (0, j)),
            pl.BlockSpec((_N_CONT, _CONT_DIM), lambda j: (0, 0)),
        ],
        out_specs=pl.BlockSpec((_N_CONT * _CONT_DIM, nb), lambda j: (0, j)),
        out_shape=jax.ShapeDtypeStruct((_N_CONT * _CONT_DIM, b), jnp.float32),
    )(xt, cont_w)


def kernel(X, cat_tables, cont_w):
    b = X.shape[0]
    xt = X.T
    t3 = jnp.transpose(cat_tables, (0, 2, 1))
    tail = jnp.pad(t3[:, :, _VMAIN:], ((0, 0), (0, 0), (0, _TAILW - _NTAIL)))
    t16 = _make_detiler()(t3, tail).reshape(_N_CAT * _VOCAB_P1, _CAT_DIM)
    out_cat_t = _make_cat_gather(b)(xt, t16)
    out_cont_t = _cont_embed_t(xt, cont_w)
    return (out_cat_t.T, out_cont_t.T)


# speedup vs baseline: 2.0689x; 2.0689x over previous
import functools

import jax
import jax.numpy as jnp
from jax import lax
from jax.experimental import pallas as pl
from jax.experimental.pallas import tpu as pltpu
from jax.experimental.pallas import tpu_sc as plsc

_N_CAT = 26
_N_CONT = 13
_VOCAB_P1 = 100001
_CAT_DIM = 16
_CONT_DIM = 16

_NC = 2
_NS = 16
_NW = _NC * _NS

_VMAIN = 99968
_NTAIL = _VOCAB_P1 - _VMAIN
_TAILW = 48


def _make_detiler():
    n_tasks = _N_CAT * 50

    mesh = plsc.VectorSubcoreMesh(core_axis_name="c", subcore_axis_name="s")

    @functools.partial(
        pl.kernel,
        mesh=mesh,
        out_type=jax.ShapeDtypeStruct((_N_CAT * _VOCAB_P1 * _CAT_DIM,),
                                      jnp.float32),
        compiler_params=pltpu.CompilerParams(
            use_tc_tiling_on_sc=True, needs_layout_passes=False),
        scratch_types=[
            pltpu.VMEM((_CAT_DIM, 2048), jnp.float32),
            pltpu.VMEM((2048 * _CAT_DIM,), jnp.float32),
            pltpu.VMEM((_CAT_DIM, _TAILW), jnp.float32),
        ],
    )
    def detile(t3_hbm, tail_hbm, out_hbm, buf, tbuf, lbuf):
        wid = lax.axis_index("s") * _NC + lax.axis_index("c")
        lane = lax.iota(jnp.int32, 16)
        rots = [jnp.bitwise_and(lane + k, 15) for k in range(16)]
        widxs = [rots[k] * _CAT_DIM + lane for k in range(16)]

        def transpose_strip(src, n_v):
            def tr_body(g, c):
                v0 = g * 16
                for k in range(16):
                    vec = plsc.load_gather(src, [lane, v0 + rots[k]])
                    plsc.store_scatter(tbuf, [v0 * _CAT_DIM + widxs[k]], vec)
                return c

            lax.fori_loop(0, n_v // 16, tr_body, 0)

        def task_body(k, carry):
            t = wid + k * _NW
            i = t // 50
            j = t % 50

            @pl.when((t < n_tasks) & (j < 48))
            def _strip():
                v0 = pl.multiple_of(j * 2048, 128)
                pltpu.sync_copy(t3_hbm.at[i, :, pl.ds(v0, 2048)], buf)
                transpose_strip(buf, 2048)
                pltpu.sync_copy(
                    tbuf,
                    out_hbm.at[pl.ds(
                        pl.multiple_of((i * _VOCAB_P1 + v0) * _CAT_DIM, 16),
                        2048 * _CAT_DIM)])

            @pl.when((t < n_tasks) & (j == 48))
            def _strip_last():
                v0 = pl.multiple_of(98304, 128)
                pltpu.sync_copy(
                    t3_hbm.at[i, :, pl.ds(v0, 1664)],
                    buf.at[:, pl.ds(0, 1664)])
                transpose_strip(buf, 1664)
                pltpu.sync_copy(
                    tbuf.at[pl.ds(0, 1664 * _CAT_DIM)],
                    out_hbm.at[pl.ds(
                        pl.multiple_of((i * _VOCAB_P1 + v0) * _CAT_DIM, 16),
                        1664 * _CAT_DIM)])

            @pl.when((t < n_tasks) & (j == 49))
            def _tail():
                pltpu.sync_copy(tail_hbm.at[i], lbuf)
                transpose_strip(lbuf, _TAILW)
                pltpu.sync_copy(
                    tbuf.at[pl.ds(0, _NTAIL * _CAT_DIM)],
                    out_hbm.at[pl.ds(
                        pl.multiple_of(
                            (i * _VOCAB_P1 + _VMAIN) * _CAT_DIM, 16),
                        _NTAIL * _CAT_DIM)])

            return carry

        lax.fori_loop(0, pl.cdiv(n_tasks, _NW), task_body, 0)

    return detile


def _make_cat_gather(b: int):
    chunk = b // _NW

    mesh = plsc.VectorSubcoreMesh(core_axis_name="c", subcore_axis_name="s")

    @functools.partial(
        pl.kernel,
        mesh=mesh,
        out_type=jax.ShapeDtypeStruct((_N_CAT * _CAT_DIM, b), jnp.float32),
        compiler_params=pltpu.CompilerParams(
            use_tc_tiling_on_sc=False, needs_layout_passes=False),
        scratch_types=[
            pltpu.VMEM((chunk,), jnp.float32),
            pltpu.VMEM((chunk,), jnp.int32),
            pltpu.VMEM((chunk, _CAT_DIM), jnp.float32),
            pltpu.VMEM((_CAT_DIM, chunk), jnp.float32),
            pltpu.SemaphoreType.DMA,
        ],
    )
    def cat_gather(xt_hbm, t16_hbm, out_hbm, idxf_v, idx_v, val_v, wbuf, sem):
        wid = lax.axis_index("s") * _NC + lax.axis_index("c")
        b0 = wid * chunk
        lane = lax.iota(jnp.int32, 16)
        rots = [jnp.bitwise_and(lane + k, 15) for k in range(16)]

        def col_body(i, carry):
            pltpu.sync_copy(xt_hbm.at[i, pl.ds(b0, chunk)], idxf_v)
            base = i * _VOCAB_P1

            def cvt_body(u, c):
                sl = pl.ds(u * 16, 16)
                idx_v[sl] = idxf_v[sl].astype(jnp.int32) + base
                return c

            lax.fori_loop(0, chunk // 16, cvt_body, 0)
            pltpu.async_copy(t16_hbm.at[idx_v], val_v, sem).wait()

            def tr_body(g, c):
                n0 = g * 16
                for k in range(_CAT_DIM):
                    vec = plsc.load_gather(val_v, [n0 + lane, rots[k]])
                    plsc.store_scatter(wbuf, [rots[k], n0 + lane], vec)
                return c

            lax.fori_loop(0, chunk // 16, tr_body, 0)
            pltpu.sync_copy(
                wbuf,
                out_hbm.at[pl.ds(i * _CAT_DIM, _CAT_DIM), pl.ds(b0, chunk)])
            return carry

        lax.fori_loop(0, _N_CAT, col_body, 0)

    return cat_gather


def _cont_body(x_ref, w_ref, o_ref):
    for s in range(_N_CONT):
        o_ref[s * _CONT_DIM:(s + 1) * _CONT_DIM, :] = (
            w_ref[s, :][:, None] * x_ref[_N_CAT + s, :][None, :]
        )


def _cont_embed_t(xt, cont_w):
    b = xt.shape[1]
    nb = 1024
    grid = (b // nb,)
    return pl.pallas_call(
        _cont_body,
        grid=grid,
        in_specs=[
            pl.BlockSpec((_N_CAT + _N_CONT, nb), lambda j: (0, j)),
            pl.BlockSpec((_N_CONT, _CONT_DIM), lambda j: (0, 0)),
        ],
        out_specs=pl.BlockSpec((_N_CONT * _CONT_DIM, nb), lambda j: (0, j)),
        out_shape=jax.ShapeDtypeStruct((_N_CONT * _CONT_DIM, b), jnp.float32),
    )(xt, cont_w)


def kernel(X, cat_tables, cont_w):
    b = X.shape[0]
    xt = X.T
    t3 = jnp.transpose(cat_tables, (0, 2, 1))
    tail = jnp.pad(t3[:, :, _VMAIN:], ((0, 0), (0, 0), (0, _TAILW - _NTAIL)))
    t16 = _make_detiler()(t3, tail).reshape(_N_CAT * _VOCAB_P1, _CAT_DIM)
    out_cat_t = _make_cat_gather(b)(xt, t16)
    out_cont_t = _cont_embed_t(xt, cont_w)
    return (out_cat_t.T, out_cont_t.T)


# speedup vs baseline: 2.6126x; 1.2628x over previous
import functools

import jax
import jax.numpy as jnp
from jax import lax
from jax.experimental import pallas as pl
from jax.experimental.pallas import tpu as pltpu
from jax.experimental.pallas import tpu_sc as plsc

_N_CAT = 26
_N_CONT = 13
_VOCAB_P1 = 100001
_CAT_DIM = 16
_CONT_DIM = 16

_NC = 2
_NS = 16
_NW = _NC * _NS

_VMAIN = 99968
_NTAIL = _VOCAB_P1 - _VMAIN
_TAILW = 48


_SW = 1024
_SLAST = _VMAIN - 97 * _SW
_TPC = 99


def _make_detiler():
    n_tasks = _N_CAT * _TPC

    mesh = plsc.VectorSubcoreMesh(core_axis_name="c", subcore_axis_name="s")

    @functools.partial(
        pl.kernel,
        mesh=mesh,
        out_type=jax.ShapeDtypeStruct((_N_CAT * _VOCAB_P1 * _CAT_DIM,),
                                      jnp.float32),
        compiler_params=pltpu.CompilerParams(
            use_tc_tiling_on_sc=True, needs_layout_passes=False),
        scratch_types=[
            pltpu.VMEM((_CAT_DIM, _SW), jnp.float32),
            pltpu.VMEM((_CAT_DIM, _SW), jnp.float32),
            pltpu.VMEM((_CAT_DIM, _TAILW), jnp.float32),
            pltpu.VMEM((_CAT_DIM, _TAILW), jnp.float32),
            pltpu.VMEM((_SW * _CAT_DIM,), jnp.float32),
            pltpu.VMEM((_SW * _CAT_DIM,), jnp.float32),
            pltpu.SemaphoreType.DMA,
            pltpu.SemaphoreType.DMA,
            pltpu.SemaphoreType.DMA,
            pltpu.SemaphoreType.DMA,
        ],
    )
    def detile(t3_hbm, tail_hbm, out_hbm, buf0, buf1, lb0, lb1, tb0, tb1,
               si0, si1, so0, so1):
        bufs, tbufs = (buf0, buf1), (tb0, tb1)
        lbufs = (lb0, lb1)
        sis, sos = (si0, si1), (so0, so1)
        wid = lax.axis_index("s") * _NC + lax.axis_index("c")
        lane = lax.iota(jnp.int32, 16)
        rots = [jnp.bitwise_and(lane + k, 15) for k in range(16)]
        widxs = [rots[k] * _CAT_DIM + lane for k in range(16)]

        def in_args(t, p):
            i = t // _TPC
            j = t % _TPC
            return i, j, bufs[p], sis[p]

        def fire_in(t, p):
            i, j, buf, sem = in_args(t, p)

            @pl.when((t < n_tasks) & (j < 97))
            def _a():
                v0 = pl.multiple_of(j * _SW, 128)
                pltpu.async_copy(t3_hbm.at[i, :, pl.ds(v0, _SW)], buf, sem)

            @pl.when((t < n_tasks) & (j == 97))
            def _b():
                v0 = pl.multiple_of(97 * _SW, 128)
                pltpu.async_copy(
                    t3_hbm.at[i, :, pl.ds(v0, _SLAST)],
                    buf.at[:, pl.ds(0, _SLAST)], sem)

            @pl.when((t < n_tasks) & (j == 98))
            def _c():
                pltpu.async_copy(tail_hbm.at[i], lbufs[p], sem)

        def wait_in(t, p):
            i, j, buf, sem = in_args(t, p)

            @pl.when((t < n_tasks) & (j < 97))
            def _a():
                v0 = pl.multiple_of(j * _SW, 128)
                pltpu.make_async_copy(
                    t3_hbm.at[i, :, pl.ds(v0, _SW)], buf, sem).wait()

            @pl.when((t < n_tasks) & (j == 97))
            def _b():
                v0 = pl.multiple_of(97 * _SW, 128)
                pltpu.make_async_copy(
                    t3_hbm.at[i, :, pl.ds(v0, _SLAST)],
                    buf.at[:, pl.ds(0, _SLAST)], sem).wait()

            @pl.when((t < n_tasks) & (j == 98))
            def _c():
                pltpu.make_async_copy(tail_hbm.at[i], lbufs[p], sem).wait()

        def out_args(t, p):
            i = t // _TPC
            j = t % _TPC
            tbuf, sem = tbufs[p], sos[p]
            v0 = jnp.where(j == 98, _VMAIN, j * _SW)
            base = pl.multiple_of((i * _VOCAB_P1 + v0) * _CAT_DIM, 16)
            width = jnp.where(
                j < 97, _SW, jnp.where(j == 97, _SLAST, _NTAIL))
            return j, tbuf, sem, base, width

        def fire_out(t, p):
            j, tbuf, sem, base, _ = out_args(t, p)

            @pl.when((t < n_tasks) & (j < 97))
            def _a():
                pltpu.async_copy(
                    tbuf, out_hbm.at[pl.ds(base, _SW * _CAT_DIM)], sem)

            @pl.when((t < n_tasks) & (j == 97))
            def _b():
                pltpu.async_copy(
                    tbuf.at[pl.ds(0, _SLAST * _CAT_DIM)],
                    out_hbm.at[pl.ds(base, _SLAST * _CAT_DIM)], sem)

            @pl.when((t < n_tasks) & (j == 98))
            def _c():
                pltpu.async_copy(
                    tbuf.at[pl.ds(0, _NTAIL * _CAT_DIM)],
                    out_hbm.at[pl.ds(base, _NTAIL * _CAT_DIM)], sem)

        def wait_out(t, p):
            j, tbuf, sem, base, _ = out_args(t, p)

            @pl.when((t < n_tasks) & (j < 97))
            def _a():
                pltpu.make_async_copy(
                    tbuf, out_hbm.at[pl.ds(base, _SW * _CAT_DIM)],
                    sem).wait()

            @pl.when((t < n_tasks) & (j == 97))
            def _b():
                pltpu.make_async_copy(
                    tbuf.at[pl.ds(0, _SLAST * _CAT_DIM)],
                    out_hbm.at[pl.ds(base, _SLAST * _CAT_DIM)], sem).wait()

            @pl.when((t < n_tasks) & (j == 98))
            def _c():
                pltpu.make_async_copy(
                    tbuf.at[pl.ds(0, _NTAIL * _CAT_DIM)],
                    out_hbm.at[pl.ds(base, _NTAIL * _CAT_DIM)], sem).wait()

        def transpose_strip(t, p):
            j = t % _TPC
            tbuf = tbufs[p]

            def mk_body(src):
                def tr_body(g, c):
                    v0 = g * 16
                    for k in range(16):
                        vec = plsc.load_gather(src, [lane, v0 + rots[k]])
                        plsc.store_scatter(
                            tbuf, [v0 * _CAT_DIM + widxs[k]], vec)
                    return c
                return tr_body

            @pl.when(j < 98)
            def _main():
                n16 = jnp.where(j < 97, _SW // 16, _SLAST // 16)
                lax.fori_loop(0, n16, mk_body(bufs[p]), 0)

            @pl.when(j == 98)
            def _tail():
                lax.fori_loop(0, _TAILW // 16, mk_body(lbufs[p]), 0)

        n_iter = (n_tasks + _NW - 1) // _NW
        fire_in(wid, 0)

        def half(k, p):
            t = wid + k * _NW
            fire_in(t + _NW, 1 - p)

            @pl.when(k >= 2)
            def _drain():
                wait_out(t - 2 * _NW, p)

            wait_in(t, p)
            transpose_strip(t, p)
            fire_out(t, p)

        def pair_body(m, carry):
            half(2 * m, 0)
            half(2 * m + 1, 1)
            return carry

        lax.fori_loop(0, (n_iter + 1) // 2, pair_body, 0)
        lastk = 2 * ((n_iter + 1) // 2) - 1
        wait_out(wid + (lastk - 1) * _NW, (lastk - 1) % 2)
        wait_out(wid + lastk * _NW, lastk % 2)

    return detile


def _make_cat_gather(b: int):
    chunk = b // _NW

    mesh = plsc.VectorSubcoreMesh(core_axis_name="c", subcore_axis_name="s")

    @functools.partial(
        pl.kernel,
        mesh=mesh,
        out_type=jax.ShapeDtypeStruct((_N_CAT * _CAT_DIM, b), jnp.float32),
        compiler_params=pltpu.CompilerParams(
            use_tc_tiling_on_sc=False, needs_layout_passes=False),
        scratch_types=[
            pltpu.VMEM((chunk,), jnp.float32),
            pltpu.VMEM((chunk,), jnp.int32),
            pltpu.VMEM((chunk, _CAT_DIM), jnp.float32),
            pltpu.VMEM((_CAT_DIM, chunk), jnp.float32),
            pltpu.SemaphoreType.DMA,
        ],
    )
    def cat_gather(xt_hbm, t16_hbm, out_hbm, idxf_v, idx_v, val_v, wbuf, sem):
        wid = lax.axis_index("s") * _NC + lax.axis_index("c")
        b0 = wid * chunk
        lane = lax.iota(jnp.int32, 16)
        rots = [jnp.bitwise_and(lane + k, 15) for k in range(16)]

        def col_body(i, carry):
            pltpu.sync_copy(xt_hbm.at[i, pl.ds(b0, chunk)], idxf_v)
            base = i * _VOCAB_P1

            def cvt_body(u, c):
                sl = pl.ds(u * 16, 16)
                idx_v[sl] = idxf_v[sl].astype(jnp.int32) + base
                return c

            lax.fori_loop(0, chunk // 16, cvt_body, 0)
            pltpu.async_copy(t16_hbm.at[idx_v], val_v, sem).wait()

            def tr_body(g, c):
                n0 = g * 16
                for k in range(_CAT_DIM):
                    vec = plsc.load_gather(val_v, [n0 + lane, rots[k]])
                    plsc.store_scatter(wbuf, [rots[k], n0 + lane], vec)
                return c

            lax.fori_loop(0, chunk // 16, tr_body, 0)
            pltpu.sync_copy(
                wbuf,
                out_hbm.at[pl.ds(i * _CAT_DIM, _CAT_DIM), pl.ds(b0, chunk)])
            return carry

        lax.fori_loop(0, _N_CAT, col_body, 0)

    return cat_gather


def _cont_body(x_ref, w_ref, o_ref):
    for s in range(_N_CONT):
        o_ref[s * _CONT_DIM:(s + 1) * _CONT_DIM, :] = (
            w_ref[s, :][:, None] * x_ref[_N_CAT + s, :][None, :]
        )


def _cont_embed_t(xt, cont_w):
    b = xt.shape[1]
    nb = 1024
    grid = (b // nb,)
    return pl.pallas_call(
        _cont_body,
        grid=grid,
        in_specs=[
            pl.BlockSpec((_N_CAT + _N_CONT, nb), lambda j: (0, j)),
            pl.BlockSpec((_N_CONT, _CONT_DIM), lambda j: (0, 0)),
        ],
        out_specs=pl.BlockSpec((_N_CONT * _CONT_DIM, nb), lambda j: (0, j)),
        out_shape=jax.ShapeDtypeStruct((_N_CONT * _CONT_DIM, b), jnp.float32),
    )(xt, cont_w)


def kernel(X, cat_tables, cont_w):
    b = X.shape[0]
    xt = X.T
    t3 = jnp.transpose(cat_tables, (0, 2, 1))
    tail = jnp.pad(t3[:, :, _VMAIN:], ((0, 0), (0, 0), (0, _TAILW - _NTAIL)))
    t16 = _make_detiler()(t3, tail).reshape(_N_CAT * _VOCAB_P1, _CAT_DIM)
    out_cat_t = _make_cat_gather(b)(xt, t16)
    out_cont_t = _cont_embed_t(xt, cont_w)
    return (out_cat_t.T, out_cont_t.T)


# speedup vs baseline: 2.9273x; 1.1205x over previous
import functools

import jax
import jax.numpy as jnp
from jax import lax
from jax.experimental import pallas as pl
from jax.experimental.pallas import tpu as pltpu
from jax.experimental.pallas import tpu_sc as plsc

_N_CAT = 26
_N_CONT = 13
_VOCAB_P1 = 100001
_CAT_DIM = 16
_CONT_DIM = 16

_NC = 2
_NS = 16
_NW = _NC * _NS

_VMAIN = 99968
_NTAIL = _VOCAB_P1 - _VMAIN
_TAILW = 48


_SW = 1024
_SLAST = _VMAIN - 97 * _SW
_TPC = 99


def _make_detiler():
    n_tasks = _N_CAT * _TPC

    mesh = plsc.VectorSubcoreMesh(core_axis_name="c", subcore_axis_name="s")

    @functools.partial(
        pl.kernel,
        mesh=mesh,
        out_type=jax.ShapeDtypeStruct((_N_CAT * _VOCAB_P1 * _CAT_DIM,),
                                      jnp.float32),
        compiler_params=pltpu.CompilerParams(
            use_tc_tiling_on_sc=True, needs_layout_passes=False),
        scratch_types=[
            pltpu.VMEM((_CAT_DIM, _SW), jnp.float32),
            pltpu.VMEM((_CAT_DIM, _SW), jnp.float32),
            pltpu.VMEM((_CAT_DIM, _TAILW), jnp.float32),
            pltpu.VMEM((_CAT_DIM, _TAILW), jnp.float32),
            pltpu.VMEM((_SW * _CAT_DIM,), jnp.float32),
            pltpu.VMEM((_SW * _CAT_DIM,), jnp.float32),
            pltpu.SemaphoreType.DMA,
            pltpu.SemaphoreType.DMA,
            pltpu.SemaphoreType.DMA,
            pltpu.SemaphoreType.DMA,
        ],
    )
    def detile(t3_hbm, tail_hbm, out_hbm, buf0, buf1, lb0, lb1, tb0, tb1,
               si0, si1, so0, so1):
        bufs, tbufs = (buf0, buf1), (tb0, tb1)
        lbufs = (lb0, lb1)
        sis, sos = (si0, si1), (so0, so1)
        wid = lax.axis_index("s") * _NC + lax.axis_index("c")
        lane = lax.iota(jnp.int32, 16)
        rots = [jnp.bitwise_and(lane + k, 15) for k in range(16)]
        widxs = [rots[k] * _CAT_DIM + lane for k in range(16)]

        def in_args(t, p):
            i = t // _TPC
            j = t % _TPC
            return i, j, bufs[p], sis[p]

        def fire_in(t, p):
            i, j, buf, sem = in_args(t, p)

            @pl.when((t < n_tasks) & (j < 97))
            def _a():
                v0 = pl.multiple_of(j * _SW, 128)
                pltpu.async_copy(t3_hbm.at[i, :, pl.ds(v0, _SW)], buf, sem)

            @pl.when((t < n_tasks) & (j == 97))
            def _b():
                v0 = pl.multiple_of(97 * _SW, 128)
                pltpu.async_copy(
                    t3_hbm.at[i, :, pl.ds(v0, _SLAST)],
                    buf.at[:, pl.ds(0, _SLAST)], sem)

            @pl.when((t < n_tasks) & (j == 98))
            def _c():
                pltpu.async_copy(tail_hbm.at[i], lbufs[p], sem)

        def wait_in(t, p):
            i, j, buf, sem = in_args(t, p)

            @pl.when((t < n_tasks) & (j < 97))
            def _a():
                v0 = pl.multiple_of(j * _SW, 128)
                pltpu.make_async_copy(
                    t3_hbm.at[i, :, pl.ds(v0, _SW)], buf, sem).wait()

            @pl.when((t < n_tasks) & (j == 97))
            def _b():
                v0 = pl.multiple_of(97 * _SW, 128)
                pltpu.make_async_copy(
                    t3_hbm.at[i, :, pl.ds(v0, _SLAST)],
                    buf.at[:, pl.ds(0, _SLAST)], sem).wait()

            @pl.when((t < n_tasks) & (j == 98))
            def _c():
                pltpu.make_async_copy(tail_hbm.at[i], lbufs[p], sem).wait()

        def out_args(t, p):
            i = t // _TPC
            j = t % _TPC
            tbuf, sem = tbufs[p], sos[p]
            v0 = jnp.where(j == 98, _VMAIN, j * _SW)
            base = pl.multiple_of((i * _VOCAB_P1 + v0) * _CAT_DIM, 16)
            width = jnp.where(
                j < 97, _SW, jnp.where(j == 97, _SLAST, _NTAIL))
            return j, tbuf, sem, base, width

        def fire_out(t, p):
            j, tbuf, sem, base, _ = out_args(t, p)

            @pl.when((t < n_tasks) & (j < 97))
            def _a():
                pltpu.async_copy(
                    tbuf, out_hbm.at[pl.ds(base, _SW * _CAT_DIM)], sem)

            @pl.when((t < n_tasks) & (j == 97))
            def _b():
                pltpu.async_copy(
                    tbuf.at[pl.ds(0, _SLAST * _CAT_DIM)],
                    out_hbm.at[pl.ds(base, _SLAST * _CAT_DIM)], sem)

            @pl.when((t < n_tasks) & (j == 98))
            def _c():
                pltpu.async_copy(
                    tbuf.at[pl.ds(0, _NTAIL * _CAT_DIM)],
                    out_hbm.at[pl.ds(base, _NTAIL * _CAT_DIM)], sem)

        def wait_out(t, p):
            j, tbuf, sem, base, _ = out_args(t, p)

            @pl.when((t < n_tasks) & (j < 97))
            def _a():
                pltpu.make_async_copy(
                    tbuf, out_hbm.at[pl.ds(base, _SW * _CAT_DIM)],
                    sem).wait()

            @pl.when((t < n_tasks) & (j == 97))
            def _b():
                pltpu.make_async_copy(
                    tbuf.at[pl.ds(0, _SLAST * _CAT_DIM)],
                    out_hbm.at[pl.ds(base, _SLAST * _CAT_DIM)], sem).wait()

            @pl.when((t < n_tasks) & (j == 98))
            def _c():
                pltpu.make_async_copy(
                    tbuf.at[pl.ds(0, _NTAIL * _CAT_DIM)],
                    out_hbm.at[pl.ds(base, _NTAIL * _CAT_DIM)], sem).wait()

        def transpose_strip(t, p):
            j = t % _TPC
            tbuf = tbufs[p]

            def mk_body(src):
                def tr_body(g, c):
                    v0 = g * 16
                    for k in range(16):
                        vec = plsc.load_gather(src, [lane, v0 + rots[k]])
                        plsc.store_scatter(
                            tbuf, [v0 * _CAT_DIM + widxs[k]], vec)
                    return c
                return tr_body

            @pl.when(j < 98)
            def _main():
                n16 = jnp.where(j < 97, _SW // 16, _SLAST // 16)
                lax.fori_loop(0, n16, mk_body(bufs[p]), 0)

            @pl.when(j == 98)
            def _tail():
                lax.fori_loop(0, _TAILW // 16, mk_body(lbufs[p]), 0)

        n_iter = (n_tasks + _NW - 1) // _NW
        fire_in(wid, 0)

        def half(k, p):
            t = wid + k * _NW
            fire_in(t + _NW, 1 - p)

            @pl.when(k >= 2)
            def _drain():
                wait_out(t - 2 * _NW, p)

            wait_in(t, p)
            transpose_strip(t, p)
            fire_out(t, p)

        def pair_body(m, carry):
            half(2 * m, 0)
            half(2 * m + 1, 1)
            return carry

        lax.fori_loop(0, (n_iter + 1) // 2, pair_body, 0)
        lastk = 2 * ((n_iter + 1) // 2) - 1
        wait_out(wid + (lastk - 1) * _NW, (lastk - 1) % 2)
        wait_out(wid + lastk * _NW, lastk % 2)

    return detile


def _make_cat_gather(b: int):
    chunk = b // _NW

    mesh = plsc.VectorSubcoreMesh(core_axis_name="c", subcore_axis_name="s")

    @functools.partial(
        pl.kernel,
        mesh=mesh,
        out_type=jax.ShapeDtypeStruct((_N_CAT * _CAT_DIM, b), jnp.float32),
        compiler_params=pltpu.CompilerParams(
            use_tc_tiling_on_sc=False, needs_layout_passes=False),
        scratch_types=(
            [pltpu.VMEM((chunk,), jnp.float32) for _ in range(2)]
            + [pltpu.VMEM((chunk,), jnp.int32) for _ in range(2)]
            + [pltpu.VMEM((chunk, _CAT_DIM), jnp.float32) for _ in range(2)]
            + [pltpu.VMEM((_CAT_DIM, chunk), jnp.float32) for _ in range(2)]
            + [pltpu.SemaphoreType.DMA for _ in range(6)]
        ),
    )
    def cat_gather(xt_hbm, t16_hbm, out_hbm, *rest):
        idxf = rest[0:2]
        idx = rest[2:4]
        val = rest[4:6]
        wbufs = rest[6:8]
        s_ix = rest[8:10]
        s_g = rest[10:12]
        s_o = rest[12:14]
        wid = lax.axis_index("s") * _NC + lax.axis_index("c")
        b0 = wid * chunk
        lane = lax.iota(jnp.int32, 16)
        rots = [jnp.bitwise_and(lane + k, 15) for k in range(16)]

        def fire_idx(i, p):
            @pl.when(i < _N_CAT)
            def _():
                pltpu.async_copy(
                    xt_hbm.at[i, pl.ds(b0, chunk)], idxf[p], s_ix[p])

        def cvt_and_gather(i, p):
            @pl.when(i < _N_CAT)
            def _():
                pltpu.make_async_copy(
                    xt_hbm.at[i, pl.ds(b0, chunk)], idxf[p], s_ix[p]).wait()
                base = i * _VOCAB_P1

                def cvt_body(u, c):
                    sl = pl.ds(u * 16, 16)
                    idx[p][sl] = idxf[p][sl].astype(jnp.int32) + base
                    return c

                lax.fori_loop(0, chunk // 16, cvt_body, 0)
                pltpu.async_copy(t16_hbm.at[idx[p]], val[p], s_g[p])

        def finish_col(i, p):
            pltpu.make_async_copy(t16_hbm.at[idx[p]], val[p], s_g[p]).wait()

            def tr_body(g, c):
                n0 = g * 16
                for k in range(_CAT_DIM):
                    vec = plsc.load_gather(val[p], [n0 + lane, rots[k]])
                    plsc.store_scatter(wbufs[p], [rots[k], n0 + lane], vec)
                return c

            lax.fori_loop(0, chunk // 16, tr_body, 0)
            pltpu.async_copy(
                wbufs[p],
                out_hbm.at[pl.ds(i * _CAT_DIM, _CAT_DIM), pl.ds(b0, chunk)],
                s_o[p])

        def wait_out(i, p):
            @pl.when((i >= 0) & (i < _N_CAT))
            def _():
                pltpu.make_async_copy(
                    wbufs[p],
                    out_hbm.at[pl.ds(i * _CAT_DIM, _CAT_DIM),
                               pl.ds(b0, chunk)], s_o[p]).wait()

        fire_idx(0, 0)
        cvt_and_gather(0, 0)
        fire_idx(1, 1)

        def half(i, p):
            cvt_and_gather(i + 1, 1 - p)
            fire_idx(i + 2, p)
            wait_out(i - 2, p)
            finish_col(i, p)

        def pair_body(m, carry):
            half(2 * m, 0)
            half(2 * m + 1, 1)
            return carry

        lax.fori_loop(0, _N_CAT // 2, pair_body, 0)
        wait_out(_N_CAT - 2, 0)
        wait_out(_N_CAT - 1, 1)

    return cat_gather


def _cont_body(x_ref, w_ref, o_ref):
    for s in range(_N_CONT):
        o_ref[s * _CONT_DIM:(s + 1) * _CONT_DIM, :] = (
            w_ref[s, :][:, None] * x_ref[_N_CAT + s, :][None, :]
        )


def _cont_embed_t(xt, cont_w):
    b = xt.shape[1]
    nb = 1024
    grid = (b // nb,)
    return pl.pallas_call(
        _cont_body,
        grid=grid,
        in_specs=[
            pl.BlockSpec((_N_CAT + _N_CONT, nb), lambda j: (0, j)),
            pl.BlockSpec((_N_CONT, _CONT_DIM), lambda j: (0, 0)),
        ],
        out_specs=pl.BlockSpec((_N_CONT * _CONT_DIM, nb), lambda j: (0, j)),
        out_shape=jax.ShapeDtypeStruct((_N_CONT * _CONT_DIM, b), jnp.float32),
    )(xt, cont_w)


def kernel(X, cat_tables, cont_w):
    b = X.shape[0]
    xt = X.T
    t3 = jnp.transpose(cat_tables, (0, 2, 1))
    tail = jnp.pad(t3[:, :, _VMAIN:], ((0, 0), (0, 0), (0, _TAILW - _NTAIL)))
    t16 = _make_detiler()(t3, tail).reshape(_N_CAT * _VOCAB_P1, _CAT_DIM)
    out_cat_t = _make_cat_gather(b)(xt, t16)
    out_cont_t = _cont_embed_t(xt, cont_w)
    return (out_cat_t.T, out_cont_t.T)


# speedup vs baseline: 2.9636x; 1.0124x over previous
import functools

import jax
import jax.numpy as jnp
from jax import lax
from jax.experimental import pallas as pl
from jax.experimental.pallas import tpu as pltpu
from jax.experimental.pallas import tpu_sc as plsc

_N_CAT = 26
_N_CONT = 13
_VOCAB_P1 = 100001
_CAT_DIM = 16
_CONT_DIM = 16

_NC = 2
_NS = 16
_NW = _NC * _NS

_VMAIN = 99968
_NTAIL = _VOCAB_P1 - _VMAIN
_TAILW = 48


_SW = 1024
_SLAST = _VMAIN - 97 * _SW
_TPC = 99


def _make_detiler():
    n_tasks = _N_CAT * _TPC

    mesh = plsc.VectorSubcoreMesh(core_axis_name="c", subcore_axis_name="s")

    @functools.partial(
        pl.kernel,
        mesh=mesh,
        out_type=jax.ShapeDtypeStruct((_N_CAT * _VOCAB_P1 * _CAT_DIM,),
                                      jnp.float32),
        compiler_params=pltpu.CompilerParams(
            use_tc_tiling_on_sc=True, needs_layout_passes=False),
        scratch_types=[
            pltpu.VMEM((_CAT_DIM, _SW), jnp.float32),
            pltpu.VMEM((_CAT_DIM, _SW), jnp.float32),
            pltpu.VMEM((_CAT_DIM, _SW), jnp.float32),
            pltpu.VMEM((_CAT_DIM, _TAILW), jnp.float32),
            pltpu.VMEM((_CAT_DIM, _TAILW), jnp.float32),
            pltpu.VMEM((_CAT_DIM, _TAILW), jnp.float32),
            pltpu.VMEM((_SW * _CAT_DIM,), jnp.float32),
            pltpu.VMEM((_SW * _CAT_DIM,), jnp.float32),
            pltpu.VMEM((_SW * _CAT_DIM,), jnp.float32),
            pltpu.SemaphoreType.DMA,
            pltpu.SemaphoreType.DMA,
            pltpu.SemaphoreType.DMA,
            pltpu.SemaphoreType.DMA,
            pltpu.SemaphoreType.DMA,
            pltpu.SemaphoreType.DMA,
        ],
    )
    def detile(t3_hbm, tail_hbm, out_hbm, buf0, buf1, buf2, lb0, lb1, lb2,
               tb0, tb1, tb2, si0, si1, si2, so0, so1, so2):
        bufs, tbufs = (buf0, buf1, buf2), (tb0, tb1, tb2)
        lbufs = (lb0, lb1, lb2)
        sis, sos = (si0, si1, si2), (so0, so1, so2)
        wid = lax.axis_index("s") * _NC + lax.axis_index("c")
        lane = lax.iota(jnp.int32, 16)
        rots = [jnp.bitwise_and(lane + k, 15) for k in range(16)]
        widxs = [rots[k] * _CAT_DIM + lane for k in range(16)]

        def in_args(t, p):
            i = t // _TPC
            j = t % _TPC
            return i, j, bufs[p], sis[p]

        def fire_in(t, p):
            i, j, buf, sem = in_args(t, p)

            @pl.when((t < n_tasks) & (j < 97))
            def _a():
                v0 = pl.multiple_of(j * _SW, 128)
                pltpu.async_copy(t3_hbm.at[i, :, pl.ds(v0, _SW)], buf, sem)

            @pl.when((t < n_tasks) & (j == 97))
            def _b():
                v0 = pl.multiple_of(97 * _SW, 128)
                pltpu.async_copy(
                    t3_hbm.at[i, :, pl.ds(v0, _SLAST)],
                    buf.at[:, pl.ds(0, _SLAST)], sem)

            @pl.when((t < n_tasks) & (j == 98))
            def _c():
                pltpu.async_copy(tail_hbm.at[i], lbufs[p], sem)

        def wait_in(t, p):
            i, j, buf, sem = in_args(t, p)

            @pl.when((t < n_tasks) & (j < 97))
            def _a():
                v0 = pl.multiple_of(j * _SW, 128)
                pltpu.make_async_copy(
                    t3_hbm.at[i, :, pl.ds(v0, _SW)], buf, sem).wait()

            @pl.when((t < n_tasks) & (j == 97))
            def _b():
                v0 = pl.multiple_of(97 * _SW, 128)
                pltpu.make_async_copy(
                    t3_hbm.at[i, :, pl.ds(v0, _SLAST)],
                    buf.at[:, pl.ds(0, _SLAST)], sem).wait()

            @pl.when((t < n_tasks) & (j == 98))
            def _c():
                pltpu.make_async_copy(tail_hbm.at[i], lbufs[p], sem).wait()

        def out_args(t, p):
            i = t // _TPC
            j = t % _TPC
            tbuf, sem = tbufs[p], sos[p]
            v0 = jnp.where(j == 98, _VMAIN, j * _SW)
            base = pl.multiple_of((i * _VOCAB_P1 + v0) * _CAT_DIM, 16)
            width = jnp.where(
                j < 97, _SW, jnp.where(j == 97, _SLAST, _NTAIL))
            return j, tbuf, sem, base, width

        def fire_out(t, p):
            j, tbuf, sem, base, _ = out_args(t, p)

            @pl.when((t < n_tasks) & (j < 97))
            def _a():
                pltpu.async_copy(
                    tbuf, out_hbm.at[pl.ds(base, _SW * _CAT_DIM)], sem)

            @pl.when((t < n_tasks) & (j == 97))
            def _b():
                pltpu.async_copy(
                    tbuf.at[pl.ds(0, _SLAST * _CAT_DIM)],
                    out_hbm.at[pl.ds(base, _SLAST * _CAT_DIM)], sem)

            @pl.when((t < n_tasks) & (j == 98))
            def _c():
                pltpu.async_copy(
                    tbuf.at[pl.ds(0, _NTAIL * _CAT_DIM)],
                    out_hbm.at[pl.ds(base, _NTAIL * _CAT_DIM)], sem)

        def wait_out(t, p):
            j, tbuf, sem, base, _ = out_args(t, p)

            @pl.when((t < n_tasks) & (j < 97))
            def _a():
                pltpu.make_async_copy(
                    tbuf, out_hbm.at[pl.ds(base, _SW * _CAT_DIM)],
                    sem).wait()

            @pl.when((t < n_tasks) & (j == 97))
            def _b():
                pltpu.make_async_copy(
                    tbuf.at[pl.ds(0, _SLAST * _CAT_DIM)],
                    out_hbm.at[pl.ds(base, _SLAST * _CAT_DIM)], sem).wait()

            @pl.when((t < n_tasks) & (j == 98))
            def _c():
                pltpu.make_async_copy(
                    tbuf.at[pl.ds(0, _NTAIL * _CAT_DIM)],
                    out_hbm.at[pl.ds(base, _NTAIL * _CAT_DIM)], sem).wait()

        def transpose_strip(t, p):
            j = t % _TPC
            tbuf = tbufs[p]

            def mk_body(src):
                def tr_body(g, c):
                    v0 = g * 16
                    for k in range(16):
                        vec = plsc.load_gather(src, [lane, v0 + rots[k]])
                        plsc.store_scatter(
                            tbuf, [v0 * _CAT_DIM + widxs[k]], vec)
                    return c
                return tr_body

            @pl.when(j < 98)
            def _main():
                n16 = jnp.where(j < 97, _SW // 16, _SLAST // 16)
                lax.fori_loop(0, n16, mk_body(bufs[p]), 0)

            @pl.when(j == 98)
            def _tail():
                lax.fori_loop(0, _TAILW // 16, mk_body(lbufs[p]), 0)

        n_iter = (n_tasks + _NW - 1) // _NW
        fire_in(wid, 0)
        fire_in(wid + _NW, 1)

        def third(k, p):
            t = wid + k * _NW
            fire_in(t + 2 * _NW, (p + 2) % 3)

            @pl.when(k >= 3)
            def _drain():
                wait_out(t - 3 * _NW, p)

            wait_in(t, p)
            transpose_strip(t, p)
            fire_out(t, p)

        def triple_body(m, carry):
            third(3 * m, 0)
            third(3 * m + 1, 1)
            third(3 * m + 2, 2)
            return carry

        n_triple = (n_iter + 2) // 3
        lax.fori_loop(0, n_triple, triple_body, 0)
        lastk = 3 * n_triple - 1
        for back in (2, 1, 0):
            wait_out(wid + (lastk - back) * _NW, (lastk - back) % 3)

    return detile


def _make_cat_gather(b: int):
    chunk = b // _NW

    mesh = plsc.VectorSubcoreMesh(core_axis_name="c", subcore_axis_name="s")

    @functools.partial(
        pl.kernel,
        mesh=mesh,
        out_type=jax.ShapeDtypeStruct((_N_CAT * _CAT_DIM, b), jnp.float32),
        compiler_params=pltpu.CompilerParams(
            use_tc_tiling_on_sc=False, needs_layout_passes=False),
        scratch_types=(
            [pltpu.VMEM((chunk,), jnp.float32) for _ in range(2)]
            + [pltpu.VMEM((chunk,), jnp.int32) for _ in range(2)]
            + [pltpu.VMEM((chunk, _CAT_DIM), jnp.float32) for _ in range(2)]
            + [pltpu.VMEM((_CAT_DIM, chunk), jnp.float32) for _ in range(2)]
            + [pltpu.SemaphoreType.DMA for _ in range(6)]
        ),
    )
    def cat_gather(xt_hbm, t16_hbm, out_hbm, *rest):
        idxf = rest[0:2]
        idx = rest[2:4]
        val = rest[4:6]
        wbufs = rest[6:8]
        s_ix = rest[8:10]
        s_g = rest[10:12]
        s_o = rest[12:14]
        wid = lax.axis_index("s") * _NC + lax.axis_index("c")
        b0 = wid * chunk
        lane = lax.iota(jnp.int32, 16)
        rots = [jnp.bitwise_and(lane + k, 15) for k in range(16)]

        def fire_idx(i, p):
            @pl.when(i < _N_CAT)
            def _():
                pltpu.async_copy(
                    xt_hbm.at[i, pl.ds(b0, chunk)], idxf[p], s_ix[p])

        def cvt_and_gather(i, p):
            @pl.when(i < _N_CAT)
            def _():
                pltpu.make_async_copy(
                    xt_hbm.at[i, pl.ds(b0, chunk)], idxf[p], s_ix[p]).wait()
                base = i * _VOCAB_P1

                def cvt_body(u, c):
                    sl = pl.ds(u * 16, 16)
                    idx[p][sl] = idxf[p][sl].astype(jnp.int32) + base
                    return c

                lax.fori_loop(0, chunk // 16, cvt_body, 0)
                pltpu.async_copy(t16_hbm.at[idx[p]], val[p], s_g[p])

        def finish_col(i, p):
            pltpu.make_async_copy(t16_hbm.at[idx[p]], val[p], s_g[p]).wait()

            def tr_body(g, c):
                n0 = g * 16
                for k in range(_CAT_DIM):
                    vec = plsc.load_gather(val[p], [n0 + lane, rots[k]])
                    plsc.store_scatter(wbufs[p], [rots[k], n0 + lane], vec)
                return c

            lax.fori_loop(0, chunk // 16, tr_body, 0)
            pltpu.async_copy(
                wbufs[p],
                out_hbm.at[pl.ds(i * _CAT_DIM, _CAT_DIM), pl.ds(b0, chunk)],
                s_o[p])

        def wait_out(i, p):
            @pl.when((i >= 0) & (i < _N_CAT))
            def _():
                pltpu.make_async_copy(
                    wbufs[p],
                    out_hbm.at[pl.ds(i * _CAT_DIM, _CAT_DIM),
                               pl.ds(b0, chunk)], s_o[p]).wait()

        fire_idx(0, 0)
        cvt_and_gather(0, 0)
        fire_idx(1, 1)

        def half(i, p):
            cvt_and_gather(i + 1, 1 - p)
            fire_idx(i + 2, p)
            wait_out(i - 2, p)
            finish_col(i, p)

        def pair_body(m, carry):
            half(2 * m, 0)
            half(2 * m + 1, 1)
            return carry

        lax.fori_loop(0, _N_CAT // 2, pair_body, 0)
        wait_out(_N_CAT - 2, 0)
        wait_out(_N_CAT - 1, 1)

    return cat_gather


def _cont_body(x_ref, w_ref, o_ref):
    for s in range(_N_CONT):
        o_ref[s * _CONT_DIM:(s + 1) * _CONT_DIM, :] = (
            w_ref[s, :][:, None] * x_ref[_N_CAT + s, :][None, :]
        )


def _cont_embed_t(xt, cont_w):
    b = xt.shape[1]
    nb = 1024
    grid = (b // nb,)
    return pl.pallas_call(
        _cont_body,
        grid=grid,
        in_specs=[
            pl.BlockSpec((_N_CAT + _N_CONT, nb), lambda j: (0, j)),
            pl.BlockSpec((_N_CONT, _CONT_DIM), lambda j: (0, 0)),
        ],
        out_specs=pl.BlockSpec((_N_CONT * _CONT_DIM, nb), lambda j: (0, j)),
        out_shape=jax.ShapeDtypeStruct((_N_CONT * _CONT_DIM, b), jnp.float32),
    )(xt, cont_w)


def kernel(X, cat_tables, cont_w):
    b = X.shape[0]
    xt = X.T
    t3 = jnp.transpose(cat_tables, (0, 2, 1))
    tail = jnp.pad(t3[:, :, _VMAIN:], ((0, 0), (0, 0), (0, _TAILW - _NTAIL)))
    t16 = _make_detiler()(t3, tail).reshape(_N_CAT * _VOCAB_P1, _CAT_DIM)
    out_cat_t = _make_cat_gather(b)(xt, t16)
    out_cont_t = _cont_embed_t(xt, cont_w)
    return (out_cat_t.T, out_cont_t.T)
